# Initial kernel scaffold; baseline (speedup 1.0000x reference)
import jax
import jax.numpy as jnp
from jax.experimental import pallas as pl


def kernel(token_ids, table, W1, b1, W2, b2):
    raise NotImplementedError("write your pallas kernel here")


# speedup vs baseline: 2.8763x; 2.8763x over previous
import functools

import jax
import jax.numpy as jnp
from jax import lax
from jax.experimental import pallas as pl
from jax.experimental.pallas import tpu as pltpu
from jax.experimental.pallas import tpu_sc as plsc

VOCAB = 1000000
EMB = 64
B = 16384
T = 200

NC = 2
NS = 16
NW = NC * NS
SEQ_PER_W = B // NW
IDS_CHUNK = 64
N_CHUNKS = SEQ_PER_W // IDS_CHUNK
G0 = 96
G1 = T - G0


def _seq_gather(table_hbm, ids_v, rows_v, sem, off):
    d0 = pltpu.async_copy(table_hbm.at[ids_v.at[pl.ds(off, G0)]],
                          rows_v.at[pl.ds(0, G0)], sem)
    d1 = pltpu.async_copy(table_hbm.at[ids_v.at[pl.ds(off + G0, G1)]],
                          rows_v.at[pl.ds(G0, G1)], sem)
    return d0, d1


def _seq_wait(table_hbm, ids_v, rows_v, sem, off):
    pltpu.make_async_copy(table_hbm.at[ids_v.at[pl.ds(off, G0)]],
                          rows_v.at[pl.ds(0, G0)], sem).wait()
    pltpu.make_async_copy(table_hbm.at[ids_v.at[pl.ds(off + G0, G1)]],
                          rows_v.at[pl.ds(G0, G1)], sem).wait()


@functools.partial(
    pl.kernel,
    out_type=jax.ShapeDtypeStruct((B, EMB), jnp.float32),
    mesh=plsc.VectorSubcoreMesh(core_axis_name="c", subcore_axis_name="s"),
    scratch_types=[
        pltpu.VMEM((IDS_CHUNK * T,), jnp.int32),
        pltpu.VMEM((T, EMB), jnp.float32),
        pltpu.VMEM((T, EMB), jnp.float32),
        pltpu.VMEM((SEQ_PER_W, EMB), jnp.float32),
        pltpu.SemaphoreType.DMA,
        pltpu.SemaphoreType.DMA,
    ],
    compiler_params=pltpu.CompilerParams(use_tc_tiling_on_sc=False),
)
def _pool(ids_hbm, table_hbm, out_hbm, ids_v, rows0, rows1, out_v, sem0,
          sem1):
    wid = lax.axis_index("s") * NC + lax.axis_index("c")
    ids_base = wid * (SEQ_PER_W * T)
    rows = (rows0, rows1)
    sems = (sem0, sem1)

    def chunk_off(s):
        return pl.multiple_of((s % IDS_CHUNK) * T, 8)

    def load_chunk(s):
        c = s // IDS_CHUNK
        pltpu.sync_copy(
            ids_hbm.at[pl.ds(pl.multiple_of(ids_base + c * (IDS_CHUNK * T), 8),
                             IDS_CHUNK * T)],
            ids_v)

    load_chunk(0)
    _seq_gather(table_hbm, ids_v, rows0, sem0, chunk_off(0))

    def accum(rows_p, s):
        def body(t, acc):
            return tuple(acc[j] + rows_p[t, pl.ds(16 * j, 16)]
                         for j in range(4))
        acc = lax.fori_loop(
            0, T, body,
            tuple(jnp.zeros((16,), jnp.float32) for _ in range(4)))
        for j in range(4):
            out_v[s, pl.ds(16 * j, 16)] = acc[j]

    def step(s, carry):
        del carry
        for p in range(2):
            sp = s + p
            q = (p + 1) % 2
            _seq_wait(table_hbm, ids_v, rows[p], sems[p], chunk_off(sp))
            @pl.when(sp + 1 < SEQ_PER_W)
            def _():
                @pl.when((sp + 1) % IDS_CHUNK == 0)
                def _():
                    load_chunk(sp + 1)
                _seq_gather(table_hbm, ids_v, rows[q], sems[q],
                            chunk_off(sp + 1))
            accum(rows[p], sp)
        return 0

    lax.fori_loop(0, SEQ_PER_W // 2, lambda i, c: step(i * 2, c), 0)
    pltpu.sync_copy(out_v,
                    out_hbm.at[pl.ds(pl.multiple_of(wid * SEQ_PER_W, 8),
                                     SEQ_PER_W)])


def _mlp_body(x_ref, w1_ref, b1_ref, w2_ref, b2_ref, o_ref):
    x = x_ref[...]
    h = lax.dot_general(x, w1_ref[...], (((1,), (1,)), ((), ())),
                        preferred_element_type=jnp.float32)
    h = jnp.maximum(h * (1.0 / T) + b1_ref[...], 0.0)
    o_ref[...] = lax.dot_general(h, w2_ref[...], (((1,), (1,)), ((), ())),
                                 preferred_element_type=jnp.float32) + b2_ref[...]


_BLK = 2048


def _mlp(sums, W1, b1, W2, b2):
    grid = B // _BLK
    return pl.pallas_call(
        _mlp_body,
        grid=(grid,),
        in_specs=[
            pl.BlockSpec((_BLK, EMB), lambda i: (i, 0)),
            pl.BlockSpec((EMB, EMB), lambda i: (0, 0)),
            pl.BlockSpec((1, EMB), lambda i: (0, 0)),
            pl.BlockSpec((EMB, EMB), lambda i: (0, 0)),
            pl.BlockSpec((1, EMB), lambda i: (0, 0)),
        ],
        out_specs=pl.BlockSpec((_BLK, EMB), lambda i: (i, 0)),
        out_shape=jax.ShapeDtypeStruct((B, EMB), jnp.float32),
    )(sums, W1, b1, W2, b2)


def kernel(token_ids, table, W1, b1, W2, b2):
    sums = _pool(token_ids.reshape(-1), table)
    return _mlp(sums, W1, b1.reshape(1, EMB), W2, b2.reshape(1, EMB))


# speedup vs baseline: 2.8765x; 1.0001x over previous
import functools

import jax
import jax.numpy as jnp
from jax import lax
from jax.experimental import pallas as pl
from jax.experimental.pallas import tpu as pltpu
from jax.experimental.pallas import tpu_sc as plsc

VOCAB = 1000000
EMB = 64
B = 16384
T = 200

NC = 2
NS = 16
NW = NC * NS
SEQ_PER_W = B // NW
IDS_CHUNK = 64
N_CHUNKS = SEQ_PER_W // IDS_CHUNK
G0 = 96
G1 = T - G0


def _seq_gather(table_hbm, ids_v, rows_v, sem, off):
    d0 = pltpu.async_copy(table_hbm.at[ids_v.at[pl.ds(off, G0)]],
                          rows_v.at[pl.ds(0, G0)], sem)
    d1 = pltpu.async_copy(table_hbm.at[ids_v.at[pl.ds(off + G0, G1)]],
                          rows_v.at[pl.ds(G0, G1)], sem)
    return d0, d1


def _seq_wait(table_hbm, ids_v, rows_v, sem, off):
    pltpu.make_async_copy(table_hbm.at[ids_v.at[pl.ds(off, G0)]],
                          rows_v.at[pl.ds(0, G0)], sem).wait()
    pltpu.make_async_copy(table_hbm.at[ids_v.at[pl.ds(off + G0, G1)]],
                          rows_v.at[pl.ds(G0, G1)], sem).wait()


@functools.partial(
    pl.kernel,
    out_type=jax.ShapeDtypeStruct((B, EMB), jnp.float32),
    mesh=plsc.VectorSubcoreMesh(core_axis_name="c", subcore_axis_name="s"),
    scratch_types=[
        pltpu.VMEM((IDS_CHUNK * T,), jnp.int32),
        pltpu.VMEM((T, EMB), jnp.float32),
        pltpu.VMEM((T, EMB), jnp.float32),
        pltpu.VMEM((SEQ_PER_W, EMB), jnp.float32),
        pltpu.SemaphoreType.DMA,
        pltpu.SemaphoreType.DMA,
    ],
    compiler_params=pltpu.CompilerParams(use_tc_tiling_on_sc=False),
)
def _pool(ids_hbm, table_hbm, out_hbm, ids_v, rows0, rows1, out_v, sem0,
          sem1):
    wid = lax.axis_index("s") * NC + lax.axis_index("c")
    ids_base = wid * (SEQ_PER_W * T)
    rows = (rows0, rows1)
    sems = (sem0, sem1)

    def chunk_off(s):
        return pl.multiple_of((s % IDS_CHUNK) * T, 8)

    def load_chunk(s):
        c = s // IDS_CHUNK
        pltpu.sync_copy(
            ids_hbm.at[pl.ds(pl.multiple_of(ids_base + c * (IDS_CHUNK * T), 8),
                             IDS_CHUNK * T)],
            ids_v)

    load_chunk(0)
    _seq_gather(table_hbm, ids_v, rows0, sem0, chunk_off(0))

    def accum(rows_p, s):
        def body(t, acc):
            return tuple(acc[j] + rows_p[t, pl.ds(16 * j, 16)]
                         for j in range(4))
        acc = lax.fori_loop(
            0, T, body,
            tuple(jnp.zeros((16,), jnp.float32) for _ in range(4)),
            unroll=10)
        for j in range(4):
            out_v[s, pl.ds(16 * j, 16)] = acc[j]

    def step(s, carry):
        del carry
        for p in range(2):
            sp = s + p
            q = (p + 1) % 2
            _seq_wait(table_hbm, ids_v, rows[p], sems[p], chunk_off(sp))
            @pl.when(sp + 1 < SEQ_PER_W)
            def _():
                @pl.when((sp + 1) % IDS_CHUNK == 0)
                def _():
                    load_chunk(sp + 1)
                _seq_gather(table_hbm, ids_v, rows[q], sems[q],
                            chunk_off(sp + 1))
            accum(rows[p], sp)
        return 0

    lax.fori_loop(0, SEQ_PER_W // 2, lambda i, c: step(i * 2, c), 0)
    pltpu.sync_copy(out_v,
                    out_hbm.at[pl.ds(pl.multiple_of(wid * SEQ_PER_W, 8),
                                     SEQ_PER_W)])


def _mlp_body(x_ref, w1_ref, b1_ref, w2_ref, b2_ref, o_ref):
    x = x_ref[...]
    h = lax.dot_general(x, w1_ref[...], (((1,), (1,)), ((), ())),
                        preferred_element_type=jnp.float32)
    h = jnp.maximum(h * (1.0 / T) + b1_ref[...], 0.0)
    o_ref[...] = lax.dot_general(h, w2_ref[...], (((1,), (1,)), ((), ())),
                                 preferred_element_type=jnp.float32) + b2_ref[...]


_BLK = 2048


def _mlp(sums, W1, b1, W2, b2):
    grid = B // _BLK
    return pl.pallas_call(
        _mlp_body,
        grid=(grid,),
        in_specs=[
            pl.BlockSpec((_BLK, EMB), lambda i: (i, 0)),
            pl.BlockSpec((EMB, EMB), lambda i: (0, 0)),
            pl.BlockSpec((1, EMB), lambda i: (0, 0)),
            pl.BlockSpec((EMB, EMB), lambda i: (0, 0)),
            pl.BlockSpec((1, EMB), lambda i: (0, 0)),
        ],
        out_specs=pl.BlockSpec((_BLK, EMB), lambda i: (i, 0)),
        out_shape=jax.ShapeDtypeStruct((B, EMB), jnp.float32),
    )(sums, W1, b1, W2, b2)


def kernel(token_ids, table, W1, b1, W2, b2):
    sums = _pool(token_ids.reshape(-1), table)
    return _mlp(sums, W1, b1.reshape(1, EMB), W2, b2.reshape(1, EMB))


# speedup vs baseline: 3.8343x; 1.3330x over previous
import functools

import jax
import jax.numpy as jnp
from jax import lax
from jax.experimental import pallas as pl
from jax.experimental.pallas import tpu as pltpu
from jax.experimental.pallas import tpu_sc as plsc

VOCAB = 1000000
EMB = 64
B = 16384
T = 200

NC = 2
NS = 16
NW = NC * NS
SEQ_PER_W = B // NW
IDS_CHUNK = 64
N_CHUNKS = SEQ_PER_W // IDS_CHUNK
G0 = 96
G1 = T - G0


def _seq_gather(table_hbm, ids_v, rows_v, sem, off):
    d0 = pltpu.async_copy(table_hbm.at[ids_v.at[pl.ds(off, G0)]],
                          rows_v.at[pl.ds(0, G0)], sem)
    d1 = pltpu.async_copy(table_hbm.at[ids_v.at[pl.ds(off + G0, G1)]],
                          rows_v.at[pl.ds(G0, G1)], sem)
    return d0, d1


def _seq_wait(table_hbm, ids_v, rows_v, sem, off):
    pltpu.make_async_copy(table_hbm.at[ids_v.at[pl.ds(off, G0)]],
                          rows_v.at[pl.ds(0, G0)], sem).wait()
    pltpu.make_async_copy(table_hbm.at[ids_v.at[pl.ds(off + G0, G1)]],
                          rows_v.at[pl.ds(G0, G1)], sem).wait()


NBUF = 4


@functools.partial(
    pl.kernel,
    out_type=jax.ShapeDtypeStruct((B, EMB), jnp.float32),
    mesh=plsc.VectorSubcoreMesh(core_axis_name="c", subcore_axis_name="s"),
    scratch_types=[
        pltpu.VMEM((IDS_CHUNK * T,), jnp.int32),
        [pltpu.VMEM((T, EMB), jnp.float32) for _ in range(NBUF)],
        pltpu.VMEM((SEQ_PER_W, EMB), jnp.float32),
        [pltpu.SemaphoreType.DMA for _ in range(NBUF)],
    ],
    compiler_params=pltpu.CompilerParams(use_tc_tiling_on_sc=False),
)
def _pool(ids_hbm, table_hbm, out_hbm, ids_v, rows, out_v, sems):
    wid = lax.axis_index("s") * NC + lax.axis_index("c")
    ids_base = wid * (SEQ_PER_W * T)

    def accum(rows_p, s):
        def body(t, acc):
            return tuple(acc[j] + rows_p[t, pl.ds(16 * j, 16)]
                         for j in range(4))
        acc = lax.fori_loop(
            0, T, body,
            tuple(jnp.zeros((16,), jnp.float32) for _ in range(4)),
            unroll=10)
        for j in range(4):
            out_v[s, pl.ds(16 * j, 16)] = acc[j]

    def chunk_body(c, carry):
        del carry
        pltpu.sync_copy(
            ids_hbm.at[pl.ds(pl.multiple_of(ids_base + c * (IDS_CHUNK * T), 8),
                             IDS_CHUNK * T)],
            ids_v)
        seq_base = c * IDS_CHUNK

        for j in range(NBUF - 1):
            _seq_gather(table_hbm, ids_v, rows[j], sems[j],
                        pl.multiple_of(j * T, 8))

        def step(i, carry):
            del carry
            for p in range(NBUF):
                j = i * NBUF + p
                _seq_wait(table_hbm, ids_v, rows[p], sems[p],
                          pl.multiple_of(j * T, 8))

                @pl.when(j + NBUF - 1 < IDS_CHUNK)
                def _():
                    _seq_gather(table_hbm, ids_v, rows[(p + NBUF - 1) % NBUF],
                                sems[(p + NBUF - 1) % NBUF],
                                pl.multiple_of((j + NBUF - 1) * T, 8))
                accum(rows[p], seq_base + j)
            return 0

        lax.fori_loop(0, IDS_CHUNK // NBUF, step, 0)
        return 0

    lax.fori_loop(0, N_CHUNKS, chunk_body, 0)
    pltpu.sync_copy(out_v,
                    out_hbm.at[pl.ds(pl.multiple_of(wid * SEQ_PER_W, 8),
                                     SEQ_PER_W)])


def _mlp_body(x_ref, w1_ref, b1_ref, w2_ref, b2_ref, o_ref):
    x = x_ref[...]
    h = lax.dot_general(x, w1_ref[...], (((1,), (1,)), ((), ())),
                        preferred_element_type=jnp.float32)
    h = jnp.maximum(h * (1.0 / T) + b1_ref[...], 0.0)
    o_ref[...] = lax.dot_general(h, w2_ref[...], (((1,), (1,)), ((), ())),
                                 preferred_element_type=jnp.float32) + b2_ref[...]


_BLK = 2048


def _mlp(sums, W1, b1, W2, b2):
    grid = B // _BLK
    return pl.pallas_call(
        _mlp_body,
        grid=(grid,),
        in_specs=[
            pl.BlockSpec((_BLK, EMB), lambda i: (i, 0)),
            pl.BlockSpec((EMB, EMB), lambda i: (0, 0)),
            pl.BlockSpec((1, EMB), lambda i: (0, 0)),
            pl.BlockSpec((EMB, EMB), lambda i: (0, 0)),
            pl.BlockSpec((1, EMB), lambda i: (0, 0)),
        ],
        out_specs=pl.BlockSpec((_BLK, EMB), lambda i: (i, 0)),
        out_shape=jax.ShapeDtypeStruct((B, EMB), jnp.float32),
    )(sums, W1, b1, W2, b2)


def kernel(token_ids, table, W1, b1, W2, b2):
    sums = _pool(token_ids.reshape(-1), table)
    return _mlp(sums, W1, b1.reshape(1, EMB), W2, b2.reshape(1, EMB))
